# baseline (speedup 1.0000x reference)
import functools

import jax
import jax.numpy as jnp
from jax import lax
from jax.experimental import pallas as pl
from jax.experimental.pallas import tpu as pltpu
from jax.experimental.pallas import tpu_sc as plsc

N_NODES = 100000
OUT_DIM = 256
MAX_SPECIES = 119

NC, NS = 2, 16
NW = NC * NS
CHUNK = 128
FULL_CHUNKS = N_NODES // CHUNK
TAIL = N_NODES - FULL_CHUNKS * CHUNK

_mesh = plsc.VectorSubcoreMesh(core_axis_name="c", subcore_axis_name="s")


CT_ROWS = 128

NREP = NW

BASE_CH = FULL_CHUNKS // NW
EXTRA = FULL_CHUNKS - BASE_CH * NW
MAX_CH = BASE_CH + 1
NBUF = 3
IDX_CAP = MAX_CH * CHUNK


@functools.partial(
    pl.kernel,
    mesh=_mesh,
    out_type=(
        jax.ShapeDtypeStruct((N_NODES, OUT_DIM), jnp.float32),
        jax.ShapeDtypeStruct((NREP * CT_ROWS, OUT_DIM), jnp.float32),
    ),
    scratch_types=[
        pltpu.VMEM((CT_ROWS,), jnp.int32),
        pltpu.VMEM((IDX_CAP,), jnp.int32),
        pltpu.VMEM((CHUNK, OUT_DIM), jnp.float32),
        pltpu.VMEM((CHUNK, OUT_DIM), jnp.float32),
        pltpu.VMEM((CHUNK, OUT_DIM), jnp.float32),
        pltpu.SemaphoreType.DMA,
        pltpu.SemaphoreType.DMA,
        pltpu.SemaphoreType.DMA,
        pltpu.SemaphoreType.DMA,
        pltpu.SemaphoreType.DMA,
        pltpu.SemaphoreType.DMA,
        pltpu.SemaphoreType.DMA,
        pltpu.SemaphoreType.DMA,
    ],
)
def _sc_embed(ns_hbm, elem_hbm, table_hbm, out_hbm, ctable_hbm,
              elem_v, idx_all, rows0, rows1, rows2,
              ct_g, ct_w, g0, g1, g2, w0, w1, w2):
    wid = lax.axis_index("s") * NC + lax.axis_index("c")
    rows = (rows0, rows1, rows2)
    gsem = (g0, g1, g2)
    wsem = (w0, w1, w2)

    elem_v[pl.ds(CT_ROWS - 16, 16)] = jnp.zeros((16,), jnp.int32)
    pltpu.sync_copy(elem_hbm, elem_v.at[pl.ds(0, MAX_SPECIES)])
    pltpu.async_copy(table_hbm.at[elem_v], rows0, ct_g).wait()
    ct_write = pltpu.async_copy(
        rows0, ctable_hbm.at[pl.ds(wid * CT_ROWS, CT_ROWS)], ct_w)

    nchunks = BASE_CH + (wid < EXTRA).astype(jnp.int32)
    start = BASE_CH * wid + jnp.minimum(wid, EXTRA)
    base_row = start * CHUNK

    pltpu.sync_copy(ns_hbm.at[pl.ds(base_row, BASE_CH * CHUNK)],
                    idx_all.at[pl.ds(0, BASE_CH * CHUNK)])

    @pl.when(wid < EXTRA)
    def _():
        pltpu.sync_copy(ns_hbm.at[pl.ds(base_row + BASE_CH * CHUNK, CHUNK)],
                        idx_all.at[pl.ds(BASE_CH * CHUNK, CHUNK)])

    off = wid * CT_ROWS
    for i in range(IDX_CAP // 16):
        idx_all[pl.ds(i * 16, 16)] = idx_all[pl.ds(i * 16, 16)] + off

    ct_write.wait()

    def issue_gather(g, b):
        return pltpu.async_copy(
            ctable_hbm.at[idx_all.at[pl.ds(g * CHUNK, CHUNK)]], rows[b], gsem[b])

    def issue_write(g, b):
        return pltpu.async_copy(
            rows[b], out_hbm.at[pl.ds((start + g) * CHUNK, CHUNK)], wsem[b])

    def drain_gather(b):
        pltpu.make_async_copy(ctable_hbm.at[pl.ds(0, CHUNK)], rows[b],
                              gsem[b]).wait()

    def drain_write(b):
        pltpu.make_async_copy(rows[b], out_hbm.at[pl.ds(0, CHUNK)],
                              wsem[b]).wait()

    issue_gather(0, 0)
    issue_gather(1, 1)
    for t in range(MAX_CH):

        @pl.when(t < nchunks)
        def _(t=t):
            drain_gather(t % NBUF)
            issue_write(t, t % NBUF)

        if t + 2 <= MAX_CH - 1:

            @pl.when(t + 2 < nchunks)
            def _(t=t):
                if t >= 1:
                    drain_write((t - 1) % NBUF)
                issue_gather(t + 2, (t + 2) % NBUF)

    for j in range(NBUF):
        drain_write(j)

    @pl.when(wid == NW - 1)
    def _():
        t0 = BASE_CH * CHUNK
        pltpu.sync_copy(ns_hbm.at[pl.ds(FULL_CHUNKS * CHUNK, TAIL)],
                        idx_all.at[pl.ds(t0, TAIL)])
        for i in range(TAIL // 16):
            idx_all[pl.ds(t0 + i * 16, 16)] = (
                idx_all[pl.ds(t0 + i * 16, 16)] + off)
        pltpu.async_copy(ctable_hbm.at[idx_all.at[pl.ds(t0, TAIL)]],
                         rows0.at[pl.ds(0, TAIL)], g0).wait()
        pltpu.sync_copy(rows0.at[pl.ds(0, TAIL)],
                        out_hbm.at[pl.ds(FULL_CHUNKS * CHUNK, TAIL)])


def kernel(node_species, element_indices, embed_table):
    out, _ = _sc_embed(node_species.astype(jnp.int32),
                       element_indices.astype(jnp.int32), embed_table)
    return out


# speedup vs baseline: 1.0078x; 1.0078x over previous
import functools

import jax
import jax.numpy as jnp
from jax import lax
from jax.experimental import pallas as pl
from jax.experimental.pallas import tpu as pltpu
from jax.experimental.pallas import tpu_sc as plsc

N_NODES = 100000
OUT_DIM = 256
MAX_SPECIES = 119

NC, NS = 2, 16
NW = NC * NS
CHUNK = 128
FULL_CHUNKS = N_NODES // CHUNK
TAIL = N_NODES - FULL_CHUNKS * CHUNK

_mesh = plsc.VectorSubcoreMesh(core_axis_name="c", subcore_axis_name="s")


CT_ROWS = 128

NREP = NW

BASE_CH = FULL_CHUNKS // NW
EXTRA = FULL_CHUNKS - BASE_CH * NW
MAX_CH = BASE_CH + 1
NBUF = 3
IDX_CAP = MAX_CH * CHUNK


@functools.partial(
    pl.kernel,
    mesh=_mesh,
    out_type=(
        jax.ShapeDtypeStruct((N_NODES, OUT_DIM), jnp.float32),
        jax.ShapeDtypeStruct((NREP * CT_ROWS, OUT_DIM), jnp.float32),
    ),
    scratch_types=[
        pltpu.VMEM((CT_ROWS,), jnp.int32),
        pltpu.VMEM((IDX_CAP,), jnp.int32),
        pltpu.VMEM((CHUNK, OUT_DIM), jnp.float32),
        pltpu.VMEM((CHUNK, OUT_DIM), jnp.float32),
        pltpu.VMEM((CHUNK, OUT_DIM), jnp.float32),
        pltpu.SemaphoreType.DMA,
        pltpu.SemaphoreType.DMA,
        pltpu.SemaphoreType.DMA,
        pltpu.SemaphoreType.DMA,
        pltpu.SemaphoreType.DMA,
        pltpu.SemaphoreType.DMA,
        pltpu.SemaphoreType.DMA,
        pltpu.SemaphoreType.DMA,
    ],
)
def _sc_embed(ns_hbm, elem_hbm, table_hbm, out_hbm, ctable_hbm,
              elem_v, idx_all, rows0, rows1, rows2,
              ct_g, ct_w, g0, g1, g2, w0, w1, w2):
    wid = lax.axis_index("s") * NC + lax.axis_index("c")
    rows = (rows0, rows1, rows2)
    gsem = (g0, g1, g2)
    wsem = (w0, w1, w2)

    elem_v[pl.ds(CT_ROWS - 16, 16)] = jnp.zeros((16,), jnp.int32)
    pltpu.sync_copy(elem_hbm, elem_v.at[pl.ds(0, MAX_SPECIES)])
    pltpu.async_copy(table_hbm.at[elem_v], rows0, ct_g).wait()
    ct_write = pltpu.async_copy(
        rows0, ctable_hbm.at[pl.ds(wid * CT_ROWS, CT_ROWS)], ct_w)

    nchunks = BASE_CH + (wid < EXTRA).astype(jnp.int32)
    start = BASE_CH * wid + jnp.minimum(wid, EXTRA)
    base_row = start * CHUNK

    pltpu.sync_copy(ns_hbm.at[pl.ds(base_row, BASE_CH * CHUNK)],
                    idx_all.at[pl.ds(0, BASE_CH * CHUNK)])

    @pl.when(wid < EXTRA)
    def _():
        pltpu.sync_copy(ns_hbm.at[pl.ds(base_row + BASE_CH * CHUNK, CHUNK)],
                        idx_all.at[pl.ds(BASE_CH * CHUNK, CHUNK)])

    off = wid * CT_ROWS
    for i in range(IDX_CAP // 16):
        idx_all[pl.ds(i * 16, 16)] = idx_all[pl.ds(i * 16, 16)] + off

    ct_write.wait()

    def issue_gather(g, b):
        return pltpu.async_copy(
            ctable_hbm.at[idx_all.at[pl.ds(g * CHUNK, CHUNK)]], rows[b], gsem[b])

    def issue_write(g, b):
        return pltpu.async_copy(
            rows[b], out_hbm.at[pl.ds((start + g) * CHUNK, CHUNK)], wsem[b])

    def drain_gather(b):
        pltpu.make_async_copy(ctable_hbm.at[pl.ds(0, CHUNK)], rows[b],
                              gsem[b]).wait()

    def drain_write(b):
        pltpu.make_async_copy(rows[b], out_hbm.at[pl.ds(0, CHUNK)],
                              wsem[b]).wait()

    for t in range(MAX_CH):

        @pl.when(t < nchunks)
        def _(t=t):
            if t >= NBUF:
                drain_write(t % NBUF)
            issue_gather(t, t % NBUF)

        if t >= 1:

            @pl.when(t - 1 < nchunks)
            def _(t=t):
                drain_gather((t - 1) % NBUF)
                issue_write(t - 1, (t - 1) % NBUF)

    @pl.when(MAX_CH - 1 < nchunks)
    def _():
        drain_gather((MAX_CH - 1) % NBUF)
        issue_write(MAX_CH - 1, (MAX_CH - 1) % NBUF)

    for j in range(NBUF):
        drain_write(j)

    @pl.when(wid == NW - 1)
    def _():
        t0 = BASE_CH * CHUNK
        pltpu.sync_copy(ns_hbm.at[pl.ds(FULL_CHUNKS * CHUNK, TAIL)],
                        idx_all.at[pl.ds(t0, TAIL)])
        for i in range(TAIL // 16):
            idx_all[pl.ds(t0 + i * 16, 16)] = (
                idx_all[pl.ds(t0 + i * 16, 16)] + off)
        pltpu.async_copy(ctable_hbm.at[idx_all.at[pl.ds(t0, TAIL)]],
                         rows0.at[pl.ds(0, TAIL)], g0).wait()
        pltpu.sync_copy(rows0.at[pl.ds(0, TAIL)],
                        out_hbm.at[pl.ds(FULL_CHUNKS * CHUNK, TAIL)])


def kernel(node_species, element_indices, embed_table):
    out, _ = _sc_embed(node_species.astype(jnp.int32),
                       element_indices.astype(jnp.int32), embed_table)
    return out
